# Initial kernel scaffold; baseline (speedup 1.0000x reference)
import jax
import jax.numpy as jnp
from jax.experimental import pallas as pl


def kernel(unit_duration_exec, source_duration_obs, residual_prev, prefix_unit_offset_prev, lengths):
    raise NotImplementedError("write your pallas kernel here")


# speedup vs baseline: 30.1503x; 30.1503x over previous
import functools

import jax
import jax.numpy as jnp
from jax import lax
from jax.experimental import pallas as pl
from jax.experimental.pallas import tpu as pltpu
from jax.experimental.pallas import tpu_sc as plsc

B = 16
U = 4096
CH = 1024
NCH = U // CH
MAGIC = 1.5 * 2 ** 23


@functools.partial(
    pl.kernel,
    out_type=jax.ShapeDtypeStruct((B, U), jnp.float32),
    mesh=plsc.VectorSubcoreMesh(core_axis_name="c", subcore_axis_name="s"),
    scratch_types=[
        pltpu.VMEM((B, CH), jnp.float32),
        pltpu.VMEM((B, CH), jnp.float32),
        pltpu.VMEM((B, CH), jnp.float32),
        pltpu.VMEM((B,), jnp.float32),
        pltpu.VMEM((B,), jnp.float32),
        pltpu.VMEM((B,), jnp.int32),
    ],
    compiler_params=pltpu.CompilerParams(use_tc_tiling_on_sc=False,
                                         needs_layout_passes=False),
)
def _scan_kernel(e_hbm, s_hbm, res_hbm, off_hbm, len_hbm, out_hbm,
                 e_v, s_v, o_v, r_v, f_v, l_v):
    @pl.when((lax.axis_index("c") == 0) & (lax.axis_index("s") == 0))
    def _():
        pltpu.sync_copy(res_hbm, r_v)
        pltpu.sync_copy(off_hbm, f_v)
        pltpu.sync_copy(len_hbm, l_v)
        res0 = r_v[...]
        off0 = f_v[...]
        lens = l_v[...]
        rows = lax.iota(jnp.int32, 16)
        magic = jnp.full((B,), MAGIC, jnp.float32)
        zero = jnp.zeros((B,), jnp.float32)
        hi = jnp.full((B,), 24.0, jnp.float32)
        lo = jnp.full((B,), -24.0, jnp.float32)

        carry = (res0, off0)
        for c in range(NCH):
            pltpu.sync_copy(e_hbm.at[:, pl.ds(c * CH, CH)], e_v)
            pltpu.sync_copy(s_hbm.at[:, pl.ds(c * CH, CH)], s_v)

            def body(t, carry, c=c):
                res, off = carry
                idxt = jnp.full((B,), t, jnp.int32)
                e = plsc.load_gather(e_v, [rows, idxt])
                src = plsc.load_gather(s_v, [rows, idxt])
                s_r = (src + magic) - magic
                value = e + res
                r = (value + magic) - magic
                dev = off + r - s_r
                dev_c = jnp.minimum(jnp.maximum(dev, lo), hi)
                adj = dev_c - off + s_r
                m = jnp.full((B,), c * CH + t, jnp.int32) < lens
                res = jnp.where(m, value - adj, res)
                off = jnp.where(m, dev_c, off)
                out = jnp.where(m, adj, zero)
                plsc.store_scatter(o_v, [rows, idxt], out)
                return res, off

            carry = lax.fori_loop(0, CH, body, carry)
            pltpu.sync_copy(o_v, out_hbm.at[:, pl.ds(c * CH, CH)])


def kernel(unit_duration_exec, source_duration_obs, residual_prev,
           prefix_unit_offset_prev, lengths):
    e = unit_duration_exec.astype(jnp.float32)
    src = source_duration_obs.astype(jnp.float32)
    res0 = residual_prev.reshape(B).astype(jnp.float32)
    off0 = prefix_unit_offset_prev.reshape(B).astype(jnp.float32)
    lens = lengths.astype(jnp.int32)
    return _scan_kernel(e, src, res0, off0, lens)


# speedup vs baseline: 43.0284x; 1.4271x over previous
import functools

import jax
import jax.numpy as jnp
from jax import lax
from jax.experimental import pallas as pl
from jax.experimental.pallas import tpu as pltpu
from jax.experimental.pallas import tpu_sc as plsc

B = 16
U = 4096
CH = 1024
NCH = U // CH
MAGIC = 1.5 * 2 ** 23


@functools.partial(
    pl.kernel,
    out_type=jax.ShapeDtypeStruct((B, U), jnp.float32),
    mesh=plsc.VectorSubcoreMesh(core_axis_name="c", subcore_axis_name="s"),
    scratch_types=[
        pltpu.VMEM((B, CH), jnp.float32),
        pltpu.VMEM((B, CH), jnp.float32),
        pltpu.VMEM((B, CH), jnp.float32),
        pltpu.VMEM((B,), jnp.float32),
        pltpu.VMEM((B,), jnp.float32),
        pltpu.VMEM((B,), jnp.int32),
    ],
    compiler_params=pltpu.CompilerParams(use_tc_tiling_on_sc=False,
                                         needs_layout_passes=False),
)
def _scan_kernel(e_hbm, s_hbm, res_hbm, off_hbm, len_hbm, out_hbm,
                 e_v, s_v, o_v, r_v, f_v, l_v):
    @pl.when((lax.axis_index("c") == 0) & (lax.axis_index("s") == 0))
    def _():
        pltpu.sync_copy(res_hbm, r_v)
        pltpu.sync_copy(off_hbm, f_v)
        pltpu.sync_copy(len_hbm, l_v)
        res0 = r_v[...]
        off0 = f_v[...]
        lens = l_v[...]
        rows = lax.iota(jnp.int32, 16)
        magic = jnp.full((B,), MAGIC, jnp.float32)
        zero = jnp.zeros((B,), jnp.float32)
        hi = jnp.full((B,), 24.0, jnp.float32)
        lo = jnp.full((B,), -24.0, jnp.float32)

        res = res0
        dcp = off0
        for c in range(NCH):
            pltpu.sync_copy(e_hbm.at[:, pl.ds(c * CH, CH)], e_v)
            pltpu.sync_copy(s_hbm.at[:, pl.ds(c * CH, CH)], s_v)

            idx0 = jnp.zeros((B,), jnp.int32)
            e_cur = plsc.load_gather(e_v, [rows, idx0])
            s_cur = plsc.load_gather(s_v, [rows, idx0])

            def body(t, carry, c=c):
                res, dcp, e, src = carry
                tn = jnp.minimum(t + 1, CH - 1)
                idxn = jnp.full((B,), tn, jnp.int32)
                e_nxt = plsc.load_gather(e_v, [rows, idxn])
                s_nxt = plsc.load_gather(s_v, [rows, idxn])
                Msr = src + magic
                s_r = Msr - magic
                u = dcp - s_r
                Mu = Msr - dcp
                value = e + res
                V = value + magic
                dev = V - Mu
                dev_c = jnp.minimum(jnp.maximum(dev, lo), hi)
                adj = dev_c - u
                res = value - adj
                m = jnp.full((B,), c * CH + t, jnp.int32) < lens
                out = jnp.where(m, adj, zero)
                idxt = jnp.full((B,), t, jnp.int32)
                plsc.store_scatter(o_v, [rows, idxt], out)
                return res, dev_c, e_nxt, s_nxt

            res, dcp, _, _ = lax.fori_loop(0, CH, body,
                                           (res, dcp, e_cur, s_cur), unroll=4)
            pltpu.sync_copy(o_v, out_hbm.at[:, pl.ds(c * CH, CH)])


def kernel(unit_duration_exec, source_duration_obs, residual_prev,
           prefix_unit_offset_prev, lengths):
    e = unit_duration_exec.astype(jnp.float32)
    src = source_duration_obs.astype(jnp.float32)
    res0 = residual_prev.reshape(B).astype(jnp.float32)
    off0 = prefix_unit_offset_prev.reshape(B).astype(jnp.float32)
    lens = lengths.astype(jnp.int32)
    return _scan_kernel(e, src, res0, off0, lens)
